# baseline (device time: 176187 ns/iter reference)
import jax
import jax.numpy as jnp
from jax import lax
from jax.experimental import pallas as pl
from jax.experimental.pallas import tpu as pltpu

B, S, H, Dh, Dr = 4, 256, 32, 128, 64
M = B * S
D = 4096
DC_SHARD = 128
SCALE = (Dh + Dr) ** -0.5


def _mm(x, w, out_dtype, bm, bn, bk, name="mm", out_scale=None):
    m, k = x.shape
    k2, n = w.shape
    assert k == k2 and m % bm == 0 and n % bn == 0 and k % bk == 0
    nk = k // bk

    def body(x_ref, w_ref, o_ref, acc_ref):
        @pl.when(pl.program_id(2) == 0)
        def _():
            acc_ref[...] = jnp.zeros_like(acc_ref)

        acc_ref[...] += jnp.dot(
            x_ref[...].astype(jnp.bfloat16),
            w_ref[...].astype(jnp.bfloat16),
            preferred_element_type=jnp.float32,
        )

        @pl.when(pl.program_id(2) == nk - 1)
        def _():
            acc = acc_ref[...]
            if out_scale is not None:
                acc = acc * out_scale
            o_ref[...] = acc.astype(o_ref.dtype)

    return pl.pallas_call(
        body,
        grid=(m // bm, n // bn, nk),
        in_specs=[
            pl.BlockSpec((bm, bk), lambda i, j, kk: (i, kk)),
            pl.BlockSpec((bk, bn), lambda i, j, kk: (kk, j)),
        ],
        out_specs=pl.BlockSpec((bm, bn), lambda i, j, kk: (i, j)),
        out_shape=jax.ShapeDtypeStruct((m, n), out_dtype),
        scratch_shapes=[pltpu.VMEM((bm, bn), jnp.float32)],
        name=name,
    )(x, w)


def _cast_c(x_flat, wdkv, bk=1024):
    nk = D // bk

    def body(x_ref, w_ref, xbf_ref, c_ref, acc_ref):
        kk = pl.program_id(0)
        xbf = x_ref[...].astype(jnp.bfloat16)
        xbf_ref[...] = xbf

        @pl.when(kk == 0)
        def _():
            acc_ref[...] = jnp.zeros_like(acc_ref)

        acc_ref[...] += jnp.dot(
            xbf, w_ref[...].astype(jnp.bfloat16),
            preferred_element_type=jnp.float32,
        )

        @pl.when(kk == nk - 1)
        def _():
            c_ref[...] = acc_ref[...].astype(jnp.bfloat16)

    return pl.pallas_call(
        body,
        grid=(nk,),
        in_specs=[
            pl.BlockSpec((M, bk), lambda k: (0, k)),
            pl.BlockSpec((bk, DC_SHARD), lambda k: (k, 0)),
        ],
        out_specs=[
            pl.BlockSpec((M, bk), lambda k: (0, k)),
            pl.BlockSpec((M, DC_SHARD), lambda k: (0, 0)),
        ],
        out_shape=[
            jax.ShapeDtypeStruct((M, D), jnp.bfloat16),
            jax.ShapeDtypeStruct((M, DC_SHARD), jnp.bfloat16),
        ],
        scratch_shapes=[pltpu.VMEM((M, DC_SHARD), jnp.float32)],
        name="cast_c",
    )(x_flat, wdkv)


def _mm_q_exchange(x_bf, wq, c_me, wuk, wuv, bn=1024, bk=1024):
    n = D
    nj, nk = n // bn, D // bk

    def body(x_ref, wqr_ref, c_ref, wuk_ref, wuv_ref,
             qr_out, c_rx, wuk_rx, wuv_rx, wuk_bf, wuv_bf,
             acc, send_sems, recv_sems):
        jj = pl.program_id(0)
        kk = pl.program_id(1)
        mx = lax.axis_index("x")
        my = lax.axis_index("y")
        mz = lax.axis_index("z")
        peer = (mx, 1 - my, mz)

        def mk_rdma(i, src, dst):
            return pltpu.make_async_remote_copy(
                src_ref=src, dst_ref=dst,
                send_sem=send_sems.at[i], recv_sem=recv_sems.at[i],
                device_id=peer, device_id_type=pl.DeviceIdType.MESH,
            )

        @pl.when((jj == 0) & (kk == 0))
        def _():
            wuk_bf[...] = wuk_ref[...].astype(jnp.bfloat16)
            wuv_bf[...] = wuv_ref[...].astype(jnp.bfloat16)

            barrier_sem = pltpu.get_barrier_semaphore()
            pl.semaphore_signal(
                barrier_sem, inc=1, device_id=peer,
                device_id_type=pl.DeviceIdType.MESH,
            )
            pl.semaphore_wait(barrier_sem, 1)

            mk_rdma(0, c_ref, c_rx).start()
            mk_rdma(1, wuk_bf, wuk_rx).start()
            mk_rdma(2, wuv_bf, wuv_rx).start()

        @pl.when(kk == 0)
        def _():
            acc[...] = jnp.zeros_like(acc)

        acc[...] += jnp.dot(
            x_ref[...], wqr_ref[...].astype(jnp.bfloat16),
            preferred_element_type=jnp.float32,
        )

        @pl.when(kk == nk - 1)
        def _():
            qr_out[...] = (acc[...] * SCALE).astype(jnp.bfloat16)

        @pl.when((jj == nj - 1) & (kk == nk - 1))
        def _():
            for i, (src, dst) in enumerate(
                [(c_ref, c_rx), (wuk_bf, wuk_rx), (wuv_bf, wuv_rx)]
            ):
                mk_rdma(i, src, dst).wait()

    full = lambda shp: pl.BlockSpec(shp, lambda j, k: (0, 0))
    return pl.pallas_call(
        body,
        grid=(nj, nk),
        in_specs=[
            pl.BlockSpec((M, bk), lambda j, k: (0, k)),
            pl.BlockSpec((bk, bn), lambda j, k: (k, j)),
            full((M, DC_SHARD)),
            full((DC_SHARD, D)),
            full((DC_SHARD, D)),
        ],
        out_specs=[
            pl.BlockSpec((M, bn), lambda j, k: (0, j)),
            full((M, DC_SHARD)),
            full((DC_SHARD, D)),
            full((DC_SHARD, D)),
            full((DC_SHARD, D)),
            full((DC_SHARD, D)),
        ],
        out_shape=[
            jax.ShapeDtypeStruct((M, n), jnp.bfloat16),
            jax.ShapeDtypeStruct((M, DC_SHARD), jnp.bfloat16),
            jax.ShapeDtypeStruct((DC_SHARD, D), jnp.bfloat16),
            jax.ShapeDtypeStruct((DC_SHARD, D), jnp.bfloat16),
            jax.ShapeDtypeStruct((DC_SHARD, D), jnp.bfloat16),
            jax.ShapeDtypeStruct((DC_SHARD, D), jnp.bfloat16),
        ],
        scratch_shapes=[
            pltpu.VMEM((M, bn), jnp.float32),
            pltpu.SemaphoreType.DMA((3,)),
            pltpu.SemaphoreType.DMA((3,)),
        ],
        compiler_params=pltpu.CompilerParams(collective_id=0),
        name="mm_q_exchange",
    )(x_bf, wq, c_me, wuk, wuv)


def _mm_kv(c_me, c_rx, wuk, wuk_rx, wuv, wuv_rx, bn=1024):
    nj = D // bn

    def body(c_ref, crx_ref, wuk_ref, wukrx_ref, wuv_ref, wuvrx_ref,
             k_ref, v_ref):
        cl = c_ref[...]
        cp = crx_ref[...]
        k_ref[...] = (
            jnp.dot(cl, wuk_ref[...].astype(jnp.bfloat16),
                    preferred_element_type=jnp.float32)
            + jnp.dot(cp, wukrx_ref[...], preferred_element_type=jnp.float32)
        ).astype(jnp.bfloat16)
        v_ref[...] = (
            jnp.dot(cl, wuv_ref[...].astype(jnp.bfloat16),
                    preferred_element_type=jnp.float32)
            + jnp.dot(cp, wuvrx_ref[...], preferred_element_type=jnp.float32)
        ).astype(jnp.bfloat16)

    full_c = pl.BlockSpec((M, DC_SHARD), lambda j: (0, 0))
    wspec = pl.BlockSpec((DC_SHARD, bn), lambda j: (0, j))
    ospec = pl.BlockSpec((M, bn), lambda j: (0, j))
    return pl.pallas_call(
        body,
        grid=(nj,),
        in_specs=[full_c, full_c, wspec, wspec, wspec, wspec],
        out_specs=[ospec, ospec],
        out_shape=[
            jax.ShapeDtypeStruct((M, D), jnp.bfloat16),
            jax.ShapeDtypeStruct((M, D), jnp.bfloat16),
        ],
        name="mm_kv",
    )(c_me, c_rx, wuk, wuk_rx, wuv, wuv_rx)


def _attention(q, c_me, c_rx, wuk_bf, wuk_rx, wuv_bf, wuv_rx, qr, kr,
               group=8):
    def body(q_ref, c_ref, crx_ref, wuk_ref, wukrx_ref, wuv_ref, wuvrx_ref,
             qr_ref, kr_ref, o_ref, sc_scr):
        cl = c_ref[...]
        cp = crx_ref[...]
        kv = (
            jnp.dot(cl, wuk_ref[...], preferred_element_type=jnp.float32)
            + jnp.dot(cp, wukrx_ref[...], preferred_element_type=jnp.float32)
        ).astype(jnp.bfloat16)
        vv = (
            jnp.dot(cl, wuv_ref[...], preferred_element_type=jnp.float32)
            + jnp.dot(cp, wuvrx_ref[...], preferred_element_type=jnp.float32)
        ).astype(jnp.bfloat16)

        qv = q_ref[...]
        qrv = qr_ref[...]
        krv = kr_ref[...]
        nt_dims = (((1,), (1,)), ((), ()))
        ones_bf = jnp.ones((S, 8), jnp.bfloat16)
        for g in range(H // group):
            for i in range(group):
                h = g * group + i
                qh = qv[:, h * Dh:(h + 1) * Dh]
                kh = kv[:, h * Dh:(h + 1) * Dh]
                qrh = qrv[:, h * Dr:(h + 1) * Dr]
                s = lax.dot_general(qh, kh, nt_dims,
                                    preferred_element_type=jnp.float32)
                s += lax.dot_general(qrh, krv, nt_dims,
                                     preferred_element_type=jnp.float32)
                sc_scr[:, i * S:(i + 1) * S] = s.astype(jnp.bfloat16)
            pg = jnp.exp(sc_scr[...])
            for i in range(group):
                h = g * group + i
                p = pg[:, i * S:(i + 1) * S]
                denom = jnp.dot(p, ones_bf,
                                preferred_element_type=jnp.float32)[:, :1]
                o = jnp.dot(p, vv[:, h * Dh:(h + 1) * Dh],
                            preferred_element_type=jnp.float32)
                o_ref[:, h * Dh:(h + 1) * Dh] = (
                    o * (1.0 / denom)
                ).astype(jnp.bfloat16)

    wfull = pl.BlockSpec((DC_SHARD, D), lambda b: (0, 0))
    return pl.pallas_call(
        body,
        grid=(B,),
        in_specs=[
            pl.BlockSpec((S, D), lambda b: (b, 0)),
            pl.BlockSpec((S, DC_SHARD), lambda b: (b, 0)),
            pl.BlockSpec((S, DC_SHARD), lambda b: (b, 0)),
            wfull,
            wfull,
            wfull,
            wfull,
            pl.BlockSpec((S, H * Dr), lambda b: (b, 0)),
            pl.BlockSpec((S, Dr), lambda b: (b, 0)),
        ],
        out_specs=pl.BlockSpec((S, D), lambda b: (b, 0)),
        out_shape=jax.ShapeDtypeStruct((M, D), jnp.bfloat16),
        scratch_shapes=[
            pltpu.VMEM((S, group * S), jnp.bfloat16),
        ],
        name="attention",
    )(q, c_me, c_rx, wuk_bf, wuk_rx, wuv_bf, wuv_rx, qr, kr)


def kernel(x, Wdkv, Wuk, Wuv, Wq, Wqr, Wkr, Wo):
    x_flat = x.reshape(M, D)

    x_bf, c = _cast_c(x_flat, Wdkv)

    q, c_rx, wuk_rx, wuv_rx, wuk_bf, wuv_bf = _mm_q_exchange(
        x_bf, Wq, c, Wuk, Wuv)

    qr = _mm(x_bf, Wqr, jnp.bfloat16, bm=1024, bn=2048, bk=512,
             name="mm_qr", out_scale=SCALE)
    kr = _mm(x_bf, Wkr, jnp.bfloat16, bm=1024, bn=64, bk=1024, name="mm_kr")

    o = _attention(q, c, c_rx, wuk_bf, wuk_rx, wuv_bf, wuv_rx, qr, kr)

    out = _mm(o, Wo, jnp.bfloat16, bm=1024, bn=1024, bk=1024, name="mm_out")
    return out.reshape(B, S, D)


# device time: 172175 ns/iter; 1.0233x vs baseline; 1.0233x over previous
import jax
import jax.numpy as jnp
from jax import lax
from jax.experimental import pallas as pl
from jax.experimental.pallas import tpu as pltpu

B, S, H, Dh, Dr = 4, 256, 32, 128, 64
M = B * S
D = 4096
DC_SHARD = 128
SCALE = (Dh + Dr) ** -0.5


def _mm(x, w, out_dtype, bm, bn, bk, name="mm", out_scale=None):
    m, k = x.shape
    k2, n = w.shape
    assert k == k2 and m % bm == 0 and n % bn == 0 and k % bk == 0
    nk = k // bk

    def body(x_ref, w_ref, o_ref, acc_ref):
        @pl.when(pl.program_id(2) == 0)
        def _():
            acc_ref[...] = jnp.zeros_like(acc_ref)

        acc_ref[...] += jnp.dot(
            x_ref[...].astype(jnp.bfloat16),
            w_ref[...].astype(jnp.bfloat16),
            preferred_element_type=jnp.float32,
        )

        @pl.when(pl.program_id(2) == nk - 1)
        def _():
            acc = acc_ref[...]
            if out_scale is not None:
                acc = acc * out_scale
            o_ref[...] = acc.astype(o_ref.dtype)

    return pl.pallas_call(
        body,
        grid=(m // bm, n // bn, nk),
        in_specs=[
            pl.BlockSpec((bm, bk), lambda i, j, kk: (i, kk)),
            pl.BlockSpec((bk, bn), lambda i, j, kk: (kk, j)),
        ],
        out_specs=pl.BlockSpec((bm, bn), lambda i, j, kk: (i, j)),
        out_shape=jax.ShapeDtypeStruct((m, n), out_dtype),
        scratch_shapes=[pltpu.VMEM((bm, bn), jnp.float32)],
        name=name,
    )(x, w)


def _cast_c(x_flat, wdkv, bk=1024):
    nk = D // bk

    def body(x_ref, w_ref, xbf_ref, c_ref, acc_ref):
        kk = pl.program_id(0)
        xbf = x_ref[...].astype(jnp.bfloat16)
        xbf_ref[...] = xbf

        @pl.when(kk == 0)
        def _():
            acc_ref[...] = jnp.zeros_like(acc_ref)

        acc_ref[...] += jnp.dot(
            xbf, w_ref[...].astype(jnp.bfloat16),
            preferred_element_type=jnp.float32,
        )

        @pl.when(kk == nk - 1)
        def _():
            c_ref[...] = acc_ref[...].astype(jnp.bfloat16)

    return pl.pallas_call(
        body,
        grid=(nk,),
        in_specs=[
            pl.BlockSpec((M, bk), lambda k: (0, k)),
            pl.BlockSpec((bk, DC_SHARD), lambda k: (k, 0)),
        ],
        out_specs=[
            pl.BlockSpec((M, bk), lambda k: (0, k)),
            pl.BlockSpec((M, DC_SHARD), lambda k: (0, 0)),
        ],
        out_shape=[
            jax.ShapeDtypeStruct((M, D), jnp.bfloat16),
            jax.ShapeDtypeStruct((M, DC_SHARD), jnp.bfloat16),
        ],
        scratch_shapes=[pltpu.VMEM((M, DC_SHARD), jnp.float32)],
        name="cast_c",
    )(x_flat, wdkv)


def _mm_q_exchange(x_bf, wq, c_me, wuk, wuv, bn=1024, bk=1024):
    n = D
    nj, nk = n // bn, D // bk

    def body(x_ref, wqr_ref, c_ref, wuk_ref, wuv_ref,
             qr_out, c_rx, wuk_rx, wuv_rx,
             acc, wuk_bf, wuv_bf, send_sems, recv_sems):
        jj = pl.program_id(0)
        kk = pl.program_id(1)
        mx = lax.axis_index("x")
        my = lax.axis_index("y")
        mz = lax.axis_index("z")
        peer = (mx, 1 - my, mz)

        def mk_rdma(i, src, dst):
            return pltpu.make_async_remote_copy(
                src_ref=src, dst_ref=dst,
                send_sem=send_sems.at[i], recv_sem=recv_sems.at[i],
                device_id=peer, device_id_type=pl.DeviceIdType.MESH,
            )

        @pl.when((jj == 0) & (kk == 0))
        def _():
            wuk_bf[...] = wuk_ref[...].astype(jnp.bfloat16)
            wuv_bf[...] = wuv_ref[...].astype(jnp.bfloat16)

            barrier_sem = pltpu.get_barrier_semaphore()
            pl.semaphore_signal(
                barrier_sem, inc=1, device_id=peer,
                device_id_type=pl.DeviceIdType.MESH,
            )
            pl.semaphore_wait(barrier_sem, 1)

            mk_rdma(0, c_ref, c_rx).start()
            mk_rdma(1, wuk_bf, wuk_rx).start()
            mk_rdma(2, wuv_bf, wuv_rx).start()

        @pl.when(kk == 0)
        def _():
            acc[...] = jnp.zeros_like(acc)

        acc[...] += jnp.dot(
            x_ref[...], wqr_ref[...].astype(jnp.bfloat16),
            preferred_element_type=jnp.float32,
        )

        @pl.when(kk == nk - 1)
        def _():
            qr_out[...] = (acc[...] * SCALE).astype(jnp.bfloat16)

        @pl.when((jj == nj - 1) & (kk == nk - 1))
        def _():
            for i, (src, dst) in enumerate(
                [(c_ref, c_rx), (wuk_bf, wuk_rx), (wuv_bf, wuv_rx)]
            ):
                mk_rdma(i, src, dst).wait()

    full = lambda shp: pl.BlockSpec(shp, lambda j, k: (0, 0))
    return pl.pallas_call(
        body,
        grid=(nj, nk),
        in_specs=[
            pl.BlockSpec((M, bk), lambda j, k: (0, k)),
            pl.BlockSpec((bk, bn), lambda j, k: (k, j)),
            full((M, DC_SHARD)),
            full((DC_SHARD, D)),
            full((DC_SHARD, D)),
        ],
        out_specs=[
            pl.BlockSpec((M, bn), lambda j, k: (0, j)),
            full((M, DC_SHARD)),
            full((DC_SHARD, D)),
            full((DC_SHARD, D)),
        ],
        out_shape=[
            jax.ShapeDtypeStruct((M, n), jnp.bfloat16),
            jax.ShapeDtypeStruct((M, DC_SHARD), jnp.bfloat16),
            jax.ShapeDtypeStruct((DC_SHARD, D), jnp.bfloat16),
            jax.ShapeDtypeStruct((DC_SHARD, D), jnp.bfloat16),
        ],
        scratch_shapes=[
            pltpu.VMEM((M, bn), jnp.float32),
            pltpu.VMEM((DC_SHARD, D), jnp.bfloat16),
            pltpu.VMEM((DC_SHARD, D), jnp.bfloat16),
            pltpu.SemaphoreType.DMA((3,)),
            pltpu.SemaphoreType.DMA((3,)),
        ],
        compiler_params=pltpu.CompilerParams(collective_id=0),
        name="mm_q_exchange",
    )(x_bf, wq, c_me, wuk, wuv)


def _attention(q, c_me, c_rx, wuk, wuk_rx, wuv, wuv_rx, qr, kr, group=8):
    def body(q_ref, c_ref, crx_ref, wuk_ref, wukrx_ref, wuv_ref, wuvrx_ref,
             qr_ref, kr_ref, o_ref, wuk_bf, wuv_bf, sc_scr):
        bb = pl.program_id(0)

        @pl.when(bb == 0)
        def _():
            wuk_bf[...] = wuk_ref[...].astype(jnp.bfloat16)
            wuv_bf[...] = wuv_ref[...].astype(jnp.bfloat16)

        cl = c_ref[...]
        cp = crx_ref[...]
        kv = (
            jnp.dot(cl, wuk_bf[...], preferred_element_type=jnp.float32)
            + jnp.dot(cp, wukrx_ref[...], preferred_element_type=jnp.float32)
        ).astype(jnp.bfloat16)
        vv = (
            jnp.dot(cl, wuv_bf[...], preferred_element_type=jnp.float32)
            + jnp.dot(cp, wuvrx_ref[...], preferred_element_type=jnp.float32)
        ).astype(jnp.bfloat16)

        qv = q_ref[...]
        qrv = qr_ref[...]
        krv = kr_ref[...]
        nt_dims = (((1,), (1,)), ((), ()))
        ones_bf = jnp.ones((S, 8), jnp.bfloat16)
        for g in range(H // group):
            for i in range(group):
                h = g * group + i
                qh = qv[:, h * Dh:(h + 1) * Dh]
                kh = kv[:, h * Dh:(h + 1) * Dh]
                qrh = qrv[:, h * Dr:(h + 1) * Dr]
                s = lax.dot_general(qh, kh, nt_dims,
                                    preferred_element_type=jnp.float32)
                s += lax.dot_general(qrh, krv, nt_dims,
                                     preferred_element_type=jnp.float32)
                sc_scr[:, i * S:(i + 1) * S] = s.astype(jnp.bfloat16)
            pg = jnp.exp(sc_scr[...])
            for i in range(group):
                h = g * group + i
                p = pg[:, i * S:(i + 1) * S]
                denom = jnp.dot(p, ones_bf,
                                preferred_element_type=jnp.float32)[:, :1]
                o = jnp.dot(p, vv[:, h * Dh:(h + 1) * Dh],
                            preferred_element_type=jnp.float32)
                o_ref[:, h * Dh:(h + 1) * Dh] = (
                    o * (1.0 / denom)
                ).astype(jnp.bfloat16)

    wfull = pl.BlockSpec((DC_SHARD, D), lambda b: (0, 0))
    return pl.pallas_call(
        body,
        grid=(B,),
        in_specs=[
            pl.BlockSpec((S, D), lambda b: (b, 0)),
            pl.BlockSpec((S, DC_SHARD), lambda b: (b, 0)),
            pl.BlockSpec((S, DC_SHARD), lambda b: (b, 0)),
            wfull,
            wfull,
            wfull,
            wfull,
            pl.BlockSpec((S, H * Dr), lambda b: (b, 0)),
            pl.BlockSpec((S, Dr), lambda b: (b, 0)),
        ],
        out_specs=pl.BlockSpec((S, D), lambda b: (b, 0)),
        out_shape=jax.ShapeDtypeStruct((M, D), jnp.bfloat16),
        scratch_shapes=[
            pltpu.VMEM((DC_SHARD, D), jnp.bfloat16),
            pltpu.VMEM((DC_SHARD, D), jnp.bfloat16),
            pltpu.VMEM((S, group * S), jnp.bfloat16),
        ],
        name="attention",
    )(q, c_me, c_rx, wuk, wuk_rx, wuv, wuv_rx, qr, kr)


def kernel(x, Wdkv, Wuk, Wuv, Wq, Wqr, Wkr, Wo):
    x_flat = x.reshape(M, D)

    x_bf, c = _cast_c(x_flat, Wdkv)

    q, c_rx, wuk_rx, wuv_rx = _mm_q_exchange(x_bf, Wq, c, Wuk, Wuv)

    qr = _mm(x_bf, Wqr, jnp.bfloat16, bm=1024, bn=2048, bk=512,
             name="mm_qr", out_scale=SCALE)
    kr = _mm(x_bf, Wkr, jnp.bfloat16, bm=1024, bn=64, bk=1024, name="mm_kr")

    o = _attention(q, c, c_rx, Wuk, wuk_rx, Wuv, wuv_rx, qr, kr)

    out = _mm(o, Wo, jnp.bfloat16, bm=1024, bn=1024, bk=1024, name="mm_out")
    return out.reshape(B, S, D)
